# baseline (speedup 1.0000x reference)
import functools

import jax
import jax.numpy as jnp
from jax import lax
from jax.experimental import pallas as pl
from jax.experimental.pallas import tpu as pltpu
from jax.experimental.pallas import tpu_sc as plsc

_B = 16384
_F = 26
_V = 100000
_D = 32

_info = plsc.get_sparse_core_info()
_NC = _info.num_cores
_NS = _info.num_subcores
_L = _info.num_lanes
_NW = _NC * _NS
_BW = _B // _NW
_CH = 128
_NCH = _BW // _CH

_mesh = plsc.VectorSubcoreMesh(core_axis_name="c", subcore_axis_name="s")


@functools.partial(
    pl.kernel,
    mesh=_mesh,
    out_type=jax.ShapeDtypeStruct((_B, _D), jnp.float32),
    compiler_params=pltpu.CompilerParams(use_tc_tiling_on_sc=False),
    scratch_types=[
        pltpu.VMEM((_F, _NCH, _CH), jnp.int32),
        pltpu.VMEM((_BW, _D), jnp.float32),
        pltpu.VMEM((_BW, _D), jnp.float32),
        pltpu.VMEM((_BW, _D), jnp.float32),
        pltpu.SemaphoreType.DMA,
        pltpu.SemaphoreType.DMA,
    ],
)
def _emb_sum(idx_hbm, tab_hbm, out_hbm, idx_v, acc, buf0, buf1, sem0, sem1):
    wid = lax.axis_index("s") * _NC + lax.axis_index("c")
    base = wid * _BW
    pltpu.sync_copy(idx_hbm.at[:, pl.ds(wid * _NCH, _NCH), :], idx_v)

    for f in range(1, _F):
        def _off(k, _, f=f):
            j = k // (_CH // _L)
            sl = pl.ds((k % (_CH // _L)) * _L, _L)
            idx_v[f, j, sl] += f * _V
            return 0
        lax.fori_loop(0, _BW // _L, _off, 0)

    bufs = (buf0, buf1)
    sems = (sem0, sem1)

    def _start(f, s):
        return [
            pltpu.async_copy(
                tab_hbm.at[idx_v.at[f, j]],
                bufs[s].at[pl.ds(j * _CH, _CH)],
                sems[s])
            for j in range(_NCH)
        ]

    cps = [None, None]
    cps[0] = _start(0, 0)
    for f in range(_F):
        s = f & 1
        for cp in cps[s]:
            cp.wait()
        if f + 1 < _F:
            cps[s ^ 1] = _start(f + 1, s ^ 1)
        buf = bufs[s]
        if f == 0:
            def _row(i, _, buf=buf):
                acc[i, pl.ds(0, _L)] = buf[i, pl.ds(0, _L)]
                acc[i, pl.ds(_L, _L)] = buf[i, pl.ds(_L, _L)]
                return 0
        else:
            def _row(i, _, buf=buf):
                acc[i, pl.ds(0, _L)] += buf[i, pl.ds(0, _L)]
                acc[i, pl.ds(_L, _L)] += buf[i, pl.ds(_L, _L)]
                return 0
        lax.fori_loop(0, _BW, _row, 0)

    pltpu.sync_copy(acc, out_hbm.at[pl.ds(base, _BW)])


def kernel(inputs, tables):
    idx_t = inputs.T.reshape(_F, _B // _CH, _CH).astype(jnp.int32)
    tab = tables.reshape(_F * _V, _D)
    return _emb_sum(idx_t, tab)
